# baseline (speedup 1.0000x reference)
import jax
import jax.numpy as jnp
from jax.experimental import pallas as pl
from jax.experimental.pallas import tpu as pltpu

N = 512
D = 128
H = 128
T = 16
B = 4
HB = 2
K = 3
C = 4
F32 = jnp.float32


def _dot(a, b):
    return jnp.dot(a.astype(jnp.bfloat16), b.astype(jnp.bfloat16),
                   preferred_element_type=F32)


def _input_contrib(x_ref, s, w_in, bias):
    x0 = x_ref[0]
    x1 = _dot(s, x0)
    x2 = 2.0 * _dot(s, x1) - x0
    a_list = []
    for bi in range(HB):
        sl = slice(bi * D, (bi + 1) * D)
        xc = jnp.concatenate([x0[:, sl], x1[:, sl], x2[:, sl]], axis=1)
        a_list.append(_dot(xc, w_in) + bias)
    return a_list


def _gru_step(a_list, s, wg, wc, state_ref, rs_ref):
    h0 = state_ref[...]
    h1 = _dot(s, h0)
    h2 = 2.0 * _dot(s, h1) - h0
    us = []
    for bi in range(HB):
        sl = slice(bi * H, (bi + 1) * H)
        xc = jnp.concatenate([h0[:, sl], h1[:, sl], h2[:, sl]], axis=1)
        g = jax.nn.sigmoid(a_list[bi][:, : 2 * H] + _dot(xc, wg))
        r, u = g[:, :H], g[:, H:]
        rs_ref[:, sl] = r * h0[:, sl]
        us.append(u)
    rs0 = rs_ref[...]
    rs1 = _dot(s, rs0)
    rs2 = 2.0 * _dot(s, rs1) - rs0
    new_states = []
    for bi in range(HB):
        sl = slice(bi * H, (bi + 1) * H)
        xc = jnp.concatenate([rs0[:, sl], rs1[:, sl], rs2[:, sl]], axis=1)
        c = jnp.tanh(a_list[bi][:, 2 * H:] + _dot(xc, wc))
        u = us[bi]
        new_states.append(u * h0[:, sl] + (1.0 - u) * c)
    return new_states


def _layer_body(x_ref, s_ref, win_ref, bin_ref, wg_ref, wc_ref,
                o_ref, state_ref, rs_ref):
    t = pl.program_id(1)

    @pl.when(t == 0)
    def _():
        state_ref[...] = jnp.zeros_like(state_ref)

    s = s_ref[...]
    a_list = _input_contrib(x_ref, s, win_ref[...], bin_ref[0])
    new_states = _gru_step(a_list, s, wg_ref[...], wc_ref[...],
                           state_ref, rs_ref)
    for bi in range(HB):
        sl = slice(bi * H, (bi + 1) * H)
        state_ref[:, sl] = new_states[bi]
        o_ref[0, :, sl] = new_states[bi]


def _layer(x, s, w_in, bias_in, wg_h, wc_h):
    return pl.pallas_call(
        _layer_body,
        grid=(B // HB, T),
        in_specs=[
            pl.BlockSpec((1, N, HB * D), lambda h, t: (t, 0, h)),
            pl.BlockSpec((N, N), lambda h, t: (0, 0)),
            pl.BlockSpec((K * D, 3 * H), lambda h, t: (0, 0)),
            pl.BlockSpec((1, 3 * H), lambda h, t: (0, 0)),
            pl.BlockSpec((K * H, 2 * H), lambda h, t: (0, 0)),
            pl.BlockSpec((K * H, H), lambda h, t: (0, 0)),
        ],
        out_specs=pl.BlockSpec((1, N, HB * H), lambda h, t: (t, 0, h)),
        out_shape=jax.ShapeDtypeStruct((T, N, B * H), F32),
        scratch_shapes=[
            pltpu.VMEM((N, HB * H), F32),
            pltpu.VMEM((N, HB * H), F32),
        ],
        compiler_params=pltpu.CompilerParams(
            dimension_semantics=("parallel", "arbitrary")),
    )(x, s, w_in, bias_in, wg_h, wc_h)


def _layer_final_body(x_ref, s_ref, win_ref, bin_ref, wg_ref, wc_ref,
                      m_ref, wfc_ref, bfc_ref,
                      o_ref, state_ref, rs_ref, last_ref):
    t = pl.program_id(1)

    @pl.when(t == 0)
    def _():
        state_ref[...] = jnp.zeros_like(state_ref)
        last_ref[...] = jnp.zeros_like(last_ref)

    s = s_ref[...]
    a_list = _input_contrib(x_ref, s, win_ref[...], bin_ref[0])
    new_states = _gru_step(a_list, s, wg_ref[...], wc_ref[...],
                           state_ref, rs_ref)
    for bi in range(HB):
        sl = slice(bi * H, (bi + 1) * H)
        state_ref[:, sl] = new_states[bi]
        mb = m_ref[0, 0, bi]
        last_ref[:, sl] = mb * new_states[bi] + (1.0 - mb) * last_ref[:, sl]

    @pl.when(t == T - 1)
    def _():
        wfc = wfc_ref[...]
        bfc = bfc_ref[0]
        for bi in range(HB):
            sl = slice(bi * H, (bi + 1) * H)
            lg = _dot(jnp.maximum(last_ref[:, sl], 0.0), wfc) + bfc
            o_ref[0, bi:bi + 1, :] = jnp.max(lg, axis=0, keepdims=True)


def _layer_final(x, s, w_in, bias_in, wg_h, wc_h, mask, wfc_pad, bfc_pad):
    return pl.pallas_call(
        _layer_final_body,
        grid=(B // HB, T),
        in_specs=[
            pl.BlockSpec((1, N, HB * D), lambda h, t: (t, 0, h)),
            pl.BlockSpec((N, N), lambda h, t: (0, 0)),
            pl.BlockSpec((K * D, 3 * H), lambda h, t: (0, 0)),
            pl.BlockSpec((1, 3 * H), lambda h, t: (0, 0)),
            pl.BlockSpec((K * H, 2 * H), lambda h, t: (0, 0)),
            pl.BlockSpec((K * H, H), lambda h, t: (0, 0)),
            pl.BlockSpec((1, 1, HB), lambda h, t: (T * h + t, 0, 0)),
            pl.BlockSpec((H, 128), lambda h, t: (0, 0)),
            pl.BlockSpec((1, 128), lambda h, t: (0, 0)),
        ],
        out_specs=pl.BlockSpec((1, HB, 128), lambda h, t: (h, 0, 0)),
        out_shape=jax.ShapeDtypeStruct((B // HB, HB, 128), F32),
        scratch_shapes=[
            pltpu.VMEM((N, HB * H), F32),
            pltpu.VMEM((N, HB * H), F32),
            pltpu.VMEM((N, HB * H), F32),
        ],
        compiler_params=pltpu.CompilerParams(
            dimension_semantics=("parallel", "arbitrary")),
    )(x, s, w_in, bias_in, wg_h, wc_h, mask, wfc_pad, bfc_pad)


def _split_weight(w, din, dout):
    wr = w.reshape(din + H, K, dout)
    w_in = wr[:din].transpose(1, 0, 2).reshape(K * din, dout)
    w_h = wr[din:].transpose(1, 0, 2).reshape(K * H, dout)
    return w_in, w_h


def kernel(input_seq, seq_lengths, supports, Wg0, bg0, Wc0, bc0,
           Wg1, bg1, Wc1, bc1, Wfc, bfc):
    s = supports[0]

    wg0_in, wg0_h = _split_weight(Wg0, D, 2 * H)
    wc0_in, wc0_h = _split_weight(Wc0, D, H)
    wg1_in, wg1_h = _split_weight(Wg1, H, 2 * H)
    wc1_in, wc1_h = _split_weight(Wc1, H, H)
    w0_in = jnp.concatenate([wg0_in, wc0_in], axis=1)
    w1_in = jnp.concatenate([wg1_in, wc1_in], axis=1)
    bias0 = jnp.concatenate([bg0, bc0]).reshape(1, 3 * H)
    bias1 = jnp.concatenate([bg1, bc1]).reshape(1, 3 * H)

    idx = jnp.clip(seq_lengths - 1, 0, T - 1).astype(jnp.int32)
    onehot = (jnp.arange(T, dtype=jnp.int32)[:, None]
              == idx[None, :]).astype(F32)
    mask = (onehot.reshape(T, B // HB, HB).transpose(1, 0, 2)
            .reshape((B // HB) * T, 1, HB))

    wfc_pad = jnp.zeros((H, 128), F32).at[:, :C].set(Wfc)
    bfc_pad = jnp.zeros((1, 128), F32).at[0, :C].set(bfc)

    x0 = input_seq.transpose(1, 2, 0, 3).reshape(T, N, B * D)
    out0 = _layer(x0, s, w0_in, bias0, wg0_h, wc0_h)
    logits_pad = _layer_final(out0, s, w1_in, bias1, wg1_h, wc1_h, mask,
                              wfc_pad, bfc_pad)
    return logits_pad.reshape(B, 128)[:, :C]


# speedup vs baseline: 1.3332x; 1.3332x over previous
import jax
import jax.numpy as jnp
from jax.experimental import pallas as pl
from jax.experimental.pallas import tpu as pltpu

N = 512
D = 128
H = 128
T = 16
B = 4
K = 3
C = 4
F32 = jnp.float32


def _dot(a, b):
    return jnp.dot(a.astype(jnp.bfloat16), b.astype(jnp.bfloat16),
                   preferred_element_type=F32)


def _input_contrib(x_ref, s, w_in, bias):
    x0 = x_ref[0]
    x1 = _dot(s, x0)
    x2 = 2.0 * _dot(s, x1) - x0
    a_list = []
    for bi in range(B):
        sl = slice(bi * D, (bi + 1) * D)
        xc = jnp.concatenate([x0[:, sl], x1[:, sl], x2[:, sl]], axis=1)
        a_list.append(_dot(xc, w_in) + bias)
    return a_list


def _gru_step(a_list, s, wg, wc, state_ref, rs_ref):
    h0 = state_ref[...]
    h1 = _dot(s, h0)
    h2 = 2.0 * _dot(s, h1) - h0
    us = []
    for bi in range(B):
        sl = slice(bi * H, (bi + 1) * H)
        xc = jnp.concatenate([h0[:, sl], h1[:, sl], h2[:, sl]], axis=1)
        g = jax.nn.sigmoid(a_list[bi][:, : 2 * H] + _dot(xc, wg))
        r, u = g[:, :H], g[:, H:]
        rs_ref[:, sl] = r * h0[:, sl]
        us.append(u)
    rs0 = rs_ref[...]
    rs1 = _dot(s, rs0)
    rs2 = 2.0 * _dot(s, rs1) - rs0
    new_states = []
    for bi in range(B):
        sl = slice(bi * H, (bi + 1) * H)
        xc = jnp.concatenate([rs0[:, sl], rs1[:, sl], rs2[:, sl]], axis=1)
        c = jnp.tanh(a_list[bi][:, 2 * H:] + _dot(xc, wc))
        u = us[bi]
        new_states.append(u * h0[:, sl] + (1.0 - u) * c)
    return new_states


def _layer_body(x_ref, s_ref, win_ref, bin_ref, wg_ref, wc_ref,
                o_ref, state_ref, rs_ref):
    t = pl.program_id(0)

    @pl.when(t == 0)
    def _():
        state_ref[...] = jnp.zeros_like(state_ref)

    s = s_ref[...]
    a_list = _input_contrib(x_ref, s, win_ref[...], bin_ref[0])
    new_states = _gru_step(a_list, s, wg_ref[...], wc_ref[...],
                           state_ref, rs_ref)
    for bi in range(B):
        sl = slice(bi * H, (bi + 1) * H)
        state_ref[:, sl] = new_states[bi]
        o_ref[0, :, sl] = new_states[bi]


def _layer(x, s, w_in, bias_in, wg_h, wc_h):
    return pl.pallas_call(
        _layer_body,
        grid=(T,),
        in_specs=[
            pl.BlockSpec((1, N, B * D), lambda t: (t, 0, 0)),
            pl.BlockSpec((N, N), lambda t: (0, 0)),
            pl.BlockSpec((K * D, 3 * H), lambda t: (0, 0)),
            pl.BlockSpec((1, 3 * H), lambda t: (0, 0)),
            pl.BlockSpec((K * H, 2 * H), lambda t: (0, 0)),
            pl.BlockSpec((K * H, H), lambda t: (0, 0)),
        ],
        out_specs=pl.BlockSpec((1, N, B * H), lambda t: (t, 0, 0)),
        out_shape=jax.ShapeDtypeStruct((T, N, B * H), F32),
        scratch_shapes=[
            pltpu.VMEM((N, B * H), F32),
            pltpu.VMEM((N, B * H), F32),
        ],
    )(x, s, w_in, bias_in, wg_h, wc_h)


def _layer_final_body(x_ref, s_ref, win_ref, bin_ref, wg_ref, wc_ref,
                      m_ref, wfc_ref, bfc_ref,
                      o_ref, state_ref, rs_ref, last_ref):
    t = pl.program_id(0)

    @pl.when(t == 0)
    def _():
        state_ref[...] = jnp.zeros_like(state_ref)
        last_ref[...] = jnp.zeros_like(last_ref)

    s = s_ref[...]
    a_list = _input_contrib(x_ref, s, win_ref[...], bin_ref[0])
    new_states = _gru_step(a_list, s, wg_ref[...], wc_ref[...],
                           state_ref, rs_ref)
    for bi in range(B):
        sl = slice(bi * H, (bi + 1) * H)
        state_ref[:, sl] = new_states[bi]
        mb = m_ref[0, 0, bi]
        last_ref[:, sl] = mb * new_states[bi] + (1.0 - mb) * last_ref[:, sl]

    @pl.when(t == T - 1)
    def _():
        wfc = wfc_ref[...]
        bfc = bfc_ref[0]
        for bi in range(B):
            sl = slice(bi * H, (bi + 1) * H)
            lg = _dot(jnp.maximum(last_ref[:, sl], 0.0), wfc) + bfc
            o_ref[bi:bi + 1, :] = jnp.max(lg, axis=0, keepdims=True)


def _layer_final(x, s, w_in, bias_in, wg_h, wc_h, mask, wfc_pad, bfc_pad):
    return pl.pallas_call(
        _layer_final_body,
        grid=(T,),
        in_specs=[
            pl.BlockSpec((1, N, B * D), lambda t: (t, 0, 0)),
            pl.BlockSpec((N, N), lambda t: (0, 0)),
            pl.BlockSpec((K * D, 3 * H), lambda t: (0, 0)),
            pl.BlockSpec((1, 3 * H), lambda t: (0, 0)),
            pl.BlockSpec((K * H, 2 * H), lambda t: (0, 0)),
            pl.BlockSpec((K * H, H), lambda t: (0, 0)),
            pl.BlockSpec((1, 1, B), lambda t: (t, 0, 0)),
            pl.BlockSpec((H, 128), lambda t: (0, 0)),
            pl.BlockSpec((1, 128), lambda t: (0, 0)),
        ],
        out_specs=pl.BlockSpec((B, 128), lambda t: (0, 0)),
        out_shape=jax.ShapeDtypeStruct((B, 128), F32),
        scratch_shapes=[
            pltpu.VMEM((N, B * H), F32),
            pltpu.VMEM((N, B * H), F32),
            pltpu.VMEM((N, B * H), F32),
        ],
    )(x, s, w_in, bias_in, wg_h, wc_h, mask, wfc_pad, bfc_pad)


def _split_weight(w, din, dout):
    wr = w.reshape(din + H, K, dout)
    w_in = wr[:din].transpose(1, 0, 2).reshape(K * din, dout)
    w_h = wr[din:].transpose(1, 0, 2).reshape(K * H, dout)
    return w_in, w_h


def kernel(input_seq, seq_lengths, supports, Wg0, bg0, Wc0, bc0,
           Wg1, bg1, Wc1, bc1, Wfc, bfc):
    s = supports[0]

    wg0_in, wg0_h = _split_weight(Wg0, D, 2 * H)
    wc0_in, wc0_h = _split_weight(Wc0, D, H)
    wg1_in, wg1_h = _split_weight(Wg1, H, 2 * H)
    wc1_in, wc1_h = _split_weight(Wc1, H, H)
    w0_in = jnp.concatenate([wg0_in, wc0_in], axis=1)
    w1_in = jnp.concatenate([wg1_in, wc1_in], axis=1)
    bias0 = jnp.concatenate([bg0, bc0]).reshape(1, 3 * H)
    bias1 = jnp.concatenate([bg1, bc1]).reshape(1, 3 * H)

    idx = jnp.clip(seq_lengths - 1, 0, T - 1).astype(jnp.int32)
    mask = (jnp.arange(T, dtype=jnp.int32)[:, None, None]
            == idx[None, None, :]).astype(F32)

    wfc_pad = jnp.zeros((H, 128), F32).at[:, :C].set(Wfc)
    bfc_pad = jnp.zeros((1, 128), F32).at[0, :C].set(bfc)

    x0 = input_seq.transpose(1, 2, 0, 3).reshape(T, N, B * D)
    out0 = _layer(x0, s, w0_in, bias0, wg0_h, wc0_h)
    logits_pad = _layer_final(out0, s, w1_in, bias1, wg1_h, wc1_h, mask,
                              wfc_pad, bfc_pad)
    return logits_pad[:, :C]


# speedup vs baseline: 1.3766x; 1.0325x over previous
import jax
import jax.numpy as jnp
from jax.experimental import pallas as pl
from jax.experimental.pallas import tpu as pltpu

N = 512
D = 128
H = 128
T = 16
B = 4
K = 3
C = 4
F32 = jnp.float32


def _dot(a, b):
    return jnp.dot(a.astype(jnp.bfloat16), b.astype(jnp.bfloat16),
                   preferred_element_type=F32)


def _input_contrib(x0, s, s2, w_in, bias):
    x1 = _dot(s, x0)
    x2 = _dot(s2, x0)
    a_list = []
    for bi in range(B):
        sl = slice(bi * D, (bi + 1) * D)
        xc = jnp.concatenate([x0[:, sl], x1[:, sl], x2[:, sl]], axis=1)
        a_list.append(_dot(xc, w_in) + bias)
    return a_list


def _gru_step(a_list, s, s2, wg, wc, state_ref):
    h0 = state_ref[...]
    h1 = _dot(s, h0)
    h2 = _dot(s2, h0)
    us = []
    rs_parts = []
    for bi in range(B):
        sl = slice(bi * H, (bi + 1) * H)
        xc = jnp.concatenate([h0[:, sl], h1[:, sl], h2[:, sl]], axis=1)
        g = jax.nn.sigmoid(a_list[bi][:, : 2 * H] + _dot(xc, wg))
        r, u = g[:, :H], g[:, H:]
        rs_parts.append(r * h0[:, sl])
        us.append(u)
    rs0 = jnp.concatenate(rs_parts, axis=1)
    rs1 = _dot(s, rs0)
    rs2 = _dot(s2, rs0)
    new_states = []
    for bi in range(B):
        sl = slice(bi * H, (bi + 1) * H)
        xc = jnp.concatenate([rs0[:, sl], rs1[:, sl], rs2[:, sl]], axis=1)
        c = jnp.tanh(a_list[bi][:, 2 * H:] + _dot(xc, wc))
        u = us[bi]
        new_states.append(u * h0[:, sl] + (1.0 - u) * c)
    return new_states


def _mega_body(x_ref, s_ref, w0_ref, b0_ref, wg0_ref, wc0_ref,
               w1_ref, b1_ref, wg1_ref, wc1_ref,
               m_ref, wfc_ref, bfc_ref,
               o_ref, s2_ref, st0_ref, st1_ref, last_ref):
    t = pl.program_id(0)

    @pl.when(t == 0)
    def _():
        st0_ref[...] = jnp.zeros_like(st0_ref)
        st1_ref[...] = jnp.zeros_like(st1_ref)
        last_ref[...] = jnp.zeros_like(last_ref)
        row = jax.lax.broadcasted_iota(jnp.int32, (N, N), 0)
        col = jax.lax.broadcasted_iota(jnp.int32, (N, N), 1)
        eye = (row == col).astype(F32)
        ss = s_ref[...]
        s2_ref[...] = 2.0 * _dot(ss, ss) - eye

    s = s_ref[...]
    s2 = s2_ref[...]

    a0 = _input_contrib(x_ref[0], s, s2, w0_ref[...], b0_ref[0])
    new0 = _gru_step(a0, s, s2, wg0_ref[...], wc0_ref[...], st0_ref)
    x1in = jnp.concatenate(new0, axis=1)
    for bi in range(B):
        st0_ref[:, bi * H:(bi + 1) * H] = new0[bi]

    a1 = _input_contrib(x1in, s, s2, w1_ref[...], b1_ref[0])
    new1 = _gru_step(a1, s, s2, wg1_ref[...], wc1_ref[...], st1_ref)
    for bi in range(B):
        sl = slice(bi * H, (bi + 1) * H)
        st1_ref[:, sl] = new1[bi]
        mb = m_ref[0, 0, bi]
        last_ref[:, sl] = mb * new1[bi] + (1.0 - mb) * last_ref[:, sl]

    @pl.when(t == T - 1)
    def _():
        wfc = wfc_ref[...]
        bfc = bfc_ref[0]
        for bi in range(B):
            sl = slice(bi * H, (bi + 1) * H)
            lg = _dot(jnp.maximum(last_ref[:, sl], 0.0), wfc) + bfc
            o_ref[bi:bi + 1, :] = jnp.max(lg, axis=0, keepdims=True)


def _mega(x, s, w0_in, bias0, wg0_h, wc0_h, w1_in, bias1, wg1_h, wc1_h,
          mask, wfc_pad, bfc_pad):
    return pl.pallas_call(
        _mega_body,
        grid=(T,),
        in_specs=[
            pl.BlockSpec((1, N, B * D), lambda t: (t, 0, 0)),
            pl.BlockSpec((N, N), lambda t: (0, 0)),
            pl.BlockSpec((K * D, 3 * H), lambda t: (0, 0)),
            pl.BlockSpec((1, 3 * H), lambda t: (0, 0)),
            pl.BlockSpec((K * H, 2 * H), lambda t: (0, 0)),
            pl.BlockSpec((K * H, H), lambda t: (0, 0)),
            pl.BlockSpec((K * H, 3 * H), lambda t: (0, 0)),
            pl.BlockSpec((1, 3 * H), lambda t: (0, 0)),
            pl.BlockSpec((K * H, 2 * H), lambda t: (0, 0)),
            pl.BlockSpec((K * H, H), lambda t: (0, 0)),
            pl.BlockSpec((1, 1, B), lambda t: (t, 0, 0)),
            pl.BlockSpec((H, 128), lambda t: (0, 0)),
            pl.BlockSpec((1, 128), lambda t: (0, 0)),
        ],
        out_specs=pl.BlockSpec((B, 128), lambda t: (0, 0)),
        out_shape=jax.ShapeDtypeStruct((B, 128), F32),
        scratch_shapes=[
            pltpu.VMEM((N, N), F32),
            pltpu.VMEM((N, B * H), F32),
            pltpu.VMEM((N, B * H), F32),
            pltpu.VMEM((N, B * H), F32),
        ],
    )(x, s, w0_in, bias0, wg0_h, wc0_h, w1_in, bias1, wg1_h, wc1_h,
      mask, wfc_pad, bfc_pad)


def _split_weight(w, din, dout):
    wr = w.reshape(din + H, K, dout)
    w_in = wr[:din].transpose(1, 0, 2).reshape(K * din, dout)
    w_h = wr[din:].transpose(1, 0, 2).reshape(K * H, dout)
    return w_in, w_h


def kernel(input_seq, seq_lengths, supports, Wg0, bg0, Wc0, bc0,
           Wg1, bg1, Wc1, bc1, Wfc, bfc):
    s = supports[0]

    wg0_in, wg0_h = _split_weight(Wg0, D, 2 * H)
    wc0_in, wc0_h = _split_weight(Wc0, D, H)
    wg1_in, wg1_h = _split_weight(Wg1, H, 2 * H)
    wc1_in, wc1_h = _split_weight(Wc1, H, H)
    w0_in = jnp.concatenate([wg0_in, wc0_in], axis=1)
    w1_in = jnp.concatenate([wg1_in, wc1_in], axis=1)
    bias0 = jnp.concatenate([bg0, bc0]).reshape(1, 3 * H)
    bias1 = jnp.concatenate([bg1, bc1]).reshape(1, 3 * H)

    idx = jnp.clip(seq_lengths - 1, 0, T - 1).astype(jnp.int32)
    mask = (jnp.arange(T, dtype=jnp.int32)[:, None]
            == idx[None, :]).astype(F32).reshape(T, 1, B)

    wfc_pad = jnp.zeros((H, 128), F32).at[:, :C].set(Wfc)
    bfc_pad = jnp.zeros((1, 128), F32).at[0, :C].set(bfc)

    x0 = input_seq.transpose(1, 2, 0, 3).reshape(T, N, B * D)
    logits_pad = _mega(x0, s, w0_in, bias0, wg0_h, wc0_h,
                       w1_in, bias1, wg1_h, wc1_h, mask, wfc_pad, bfc_pad)
    return logits_pad[:, :C]


# speedup vs baseline: 1.4549x; 1.0569x over previous
import jax
import jax.numpy as jnp
from jax.experimental import pallas as pl
from jax.experimental.pallas import tpu as pltpu

N = 512
D = 128
H = 128
T = 16
B = 4
K = 3
C = 4
F32 = jnp.float32
BF16 = jnp.bfloat16


def _dot(a, b):
    return jnp.dot(a.astype(BF16), b.astype(BF16),
                   preferred_element_type=F32)


def _input_contrib(x0, s, s2, w_in, bias):
    x0 = x0.astype(BF16)
    x1 = _dot(s, x0).astype(BF16)
    x2 = _dot(s2, x0).astype(BF16)
    a_list = []
    for bi in range(B):
        sl = slice(bi * D, (bi + 1) * D)
        xc = jnp.concatenate([x0[:, sl], x1[:, sl], x2[:, sl]], axis=1)
        a_list.append(_dot(xc, w_in) + bias)
    return a_list


def _gru_step(a_list, s, s2, wg, wc, state_ref):
    h0 = state_ref[...]
    h0b = h0.astype(BF16)
    h1 = _dot(s, h0b).astype(BF16)
    h2 = _dot(s2, h0b).astype(BF16)
    us = []
    rs_parts = []
    for bi in range(B):
        sl = slice(bi * H, (bi + 1) * H)
        xc = jnp.concatenate([h0b[:, sl], h1[:, sl], h2[:, sl]], axis=1)
        g = jax.nn.sigmoid(a_list[bi][:, : 2 * H] + _dot(xc, wg))
        r, u = g[:, :H], g[:, H:]
        rs_parts.append((r * h0[:, sl]).astype(BF16))
        us.append(u)
    rs0 = jnp.concatenate(rs_parts, axis=1)
    rs1 = _dot(s, rs0).astype(BF16)
    rs2 = _dot(s2, rs0).astype(BF16)
    new_states = []
    for bi in range(B):
        sl = slice(bi * H, (bi + 1) * H)
        xc = jnp.concatenate([rs0[:, sl], rs1[:, sl], rs2[:, sl]], axis=1)
        c = jnp.tanh(a_list[bi][:, 2 * H:] + _dot(xc, wc))
        u = us[bi]
        new_states.append(u * h0[:, sl] + (1.0 - u) * c)
    return new_states


def _mega_body(x_ref, s_ref, w0_ref, b0_ref, wg0_ref, wc0_ref,
               w1_ref, b1_ref, wg1_ref, wc1_ref,
               m_ref, wfc_ref, bfc_ref,
               o_ref, s2_ref, st0_ref, st1_ref, last_ref):
    t = pl.program_id(0)

    @pl.when(t == 0)
    def _():
        st0_ref[...] = jnp.zeros_like(st0_ref)
        st1_ref[...] = jnp.zeros_like(st1_ref)
        last_ref[...] = jnp.zeros_like(last_ref)
        row = jax.lax.broadcasted_iota(jnp.int32, (N, N), 0)
        col = jax.lax.broadcasted_iota(jnp.int32, (N, N), 1)
        eye = (row == col).astype(F32)
        ss = s_ref[...]
        s2_ref[...] = (2.0 * _dot(ss, ss) - eye).astype(BF16)

    s = s_ref[...]
    s2 = s2_ref[...]

    a0 = _input_contrib(x_ref[0], s, s2, w0_ref[...], b0_ref[0])
    new0 = _gru_step(a0, s, s2, wg0_ref[...], wc0_ref[...], st0_ref)
    x1in = jnp.concatenate(new0, axis=1)
    for bi in range(B):
        st0_ref[:, bi * H:(bi + 1) * H] = new0[bi]

    a1 = _input_contrib(x1in, s, s2, w1_ref[...], b1_ref[0])
    new1 = _gru_step(a1, s, s2, wg1_ref[...], wc1_ref[...], st1_ref)
    for bi in range(B):
        sl = slice(bi * H, (bi + 1) * H)
        st1_ref[:, sl] = new1[bi]
        mb = m_ref[0, 0, bi]
        last_ref[:, sl] = mb * new1[bi] + (1.0 - mb) * last_ref[:, sl]

    @pl.when(t == T - 1)
    def _():
        wfc = wfc_ref[...]
        bfc = bfc_ref[0]
        for bi in range(B):
            sl = slice(bi * H, (bi + 1) * H)
            lg = _dot(jnp.maximum(last_ref[:, sl], 0.0), wfc) + bfc
            o_ref[bi:bi + 1, :] = jnp.max(lg, axis=0, keepdims=True)


def _mega(x, s, w0_in, bias0, wg0_h, wc0_h, w1_in, bias1, wg1_h, wc1_h,
          mask, wfc_pad, bfc_pad):
    return pl.pallas_call(
        _mega_body,
        grid=(T,),
        in_specs=[
            pl.BlockSpec((1, N, B * D), lambda t: (t, 0, 0)),
            pl.BlockSpec((N, N), lambda t: (0, 0)),
            pl.BlockSpec((K * D, 3 * H), lambda t: (0, 0)),
            pl.BlockSpec((1, 3 * H), lambda t: (0, 0)),
            pl.BlockSpec((K * H, 2 * H), lambda t: (0, 0)),
            pl.BlockSpec((K * H, H), lambda t: (0, 0)),
            pl.BlockSpec((K * H, 3 * H), lambda t: (0, 0)),
            pl.BlockSpec((1, 3 * H), lambda t: (0, 0)),
            pl.BlockSpec((K * H, 2 * H), lambda t: (0, 0)),
            pl.BlockSpec((K * H, H), lambda t: (0, 0)),
            pl.BlockSpec((1, 1, B), lambda t: (t, 0, 0)),
            pl.BlockSpec((H, 128), lambda t: (0, 0)),
            pl.BlockSpec((1, 128), lambda t: (0, 0)),
        ],
        out_specs=pl.BlockSpec((B, 128), lambda t: (0, 0)),
        out_shape=jax.ShapeDtypeStruct((B, 128), F32),
        scratch_shapes=[
            pltpu.VMEM((N, N), BF16),
            pltpu.VMEM((N, B * H), F32),
            pltpu.VMEM((N, B * H), F32),
            pltpu.VMEM((N, B * H), F32),
        ],
    )(x, s, w0_in, bias0, wg0_h, wc0_h, w1_in, bias1, wg1_h, wc1_h,
      mask, wfc_pad, bfc_pad)


def _split_weight(w, din, dout):
    wr = w.reshape(din + H, K, dout)
    w_in = wr[:din].transpose(1, 0, 2).reshape(K * din, dout)
    w_h = wr[din:].transpose(1, 0, 2).reshape(K * H, dout)
    return w_in, w_h


def kernel(input_seq, seq_lengths, supports, Wg0, bg0, Wc0, bc0,
           Wg1, bg1, Wc1, bc1, Wfc, bfc):
    s = supports[0].astype(BF16)

    wg0_in, wg0_h = _split_weight(Wg0, D, 2 * H)
    wc0_in, wc0_h = _split_weight(Wc0, D, H)
    wg1_in, wg1_h = _split_weight(Wg1, H, 2 * H)
    wc1_in, wc1_h = _split_weight(Wc1, H, H)
    w0_in = jnp.concatenate([wg0_in, wc0_in], axis=1).astype(BF16)
    w1_in = jnp.concatenate([wg1_in, wc1_in], axis=1).astype(BF16)
    wg0_h = wg0_h.astype(BF16)
    wc0_h = wc0_h.astype(BF16)
    wg1_h = wg1_h.astype(BF16)
    wc1_h = wc1_h.astype(BF16)
    bias0 = jnp.concatenate([bg0, bc0]).reshape(1, 3 * H)
    bias1 = jnp.concatenate([bg1, bc1]).reshape(1, 3 * H)

    idx = jnp.clip(seq_lengths - 1, 0, T - 1).astype(jnp.int32)
    mask = (jnp.arange(T, dtype=jnp.int32)[:, None]
            == idx[None, :]).astype(F32).reshape(T, 1, B)

    wfc_pad = jnp.zeros((H, 128), BF16).at[:, :C].set(Wfc.astype(BF16))
    bfc_pad = jnp.zeros((1, 128), F32).at[0, :C].set(bfc)

    x0 = input_seq.transpose(1, 2, 0, 3).reshape(T, N, B * D).astype(BF16)
    logits_pad = _mega(x0, s, w0_in, bias0, wg0_h, wc0_h,
                       w1_in, bias1, wg1_h, wc1_h, mask, wfc_pad, bfc_pad)
    return logits_pad[:, :C]


# speedup vs baseline: 1.4577x; 1.0019x over previous
import jax
import jax.numpy as jnp
from jax.experimental import pallas as pl
from jax.experimental.pallas import tpu as pltpu

N = 512
D = 128
H = 128
T = 16
B = 4
K = 3
C = 4
F32 = jnp.float32
BF16 = jnp.bfloat16


def _dot(a, b):
    return jnp.dot(a.astype(BF16), b.astype(BF16),
                   preferred_element_type=F32)


def _proj_a(x0, x1, x2, w_in, bias, d):
    a_list = []
    for bi in range(B):
        sl = slice(bi * d, (bi + 1) * d)
        xc = jnp.concatenate([x0[:, sl], x1[:, sl], x2[:, sl]], axis=1)
        a_list.append(_dot(xc, w_in) + bias)
    return a_list


def _gru_tail(a_list, sstack, wg, wc, h0, h0b, h1, h2):
    us = []
    rs_parts = []
    for bi in range(B):
        sl = slice(bi * H, (bi + 1) * H)
        xc = jnp.concatenate([h0b[:, sl], h1[:, sl], h2[:, sl]], axis=1)
        g = jax.nn.sigmoid(a_list[bi][:, : 2 * H] + _dot(xc, wg))
        r, u = g[:, :H], g[:, H:]
        rs_parts.append((r * h0[:, sl]).astype(BF16))
        us.append(u)
    rs0 = jnp.concatenate(rs_parts, axis=1)
    rsd = _dot(sstack, rs0)
    rs1 = rsd[:N].astype(BF16)
    rs2 = rsd[N:].astype(BF16)
    new_states = []
    for bi in range(B):
        sl = slice(bi * H, (bi + 1) * H)
        xc = jnp.concatenate([rs0[:, sl], rs1[:, sl], rs2[:, sl]], axis=1)
        c = jnp.tanh(a_list[bi][:, 2 * H:] + _dot(xc, wc))
        u = us[bi]
        new_states.append(u * h0[:, sl] + (1.0 - u) * c)
    return new_states


def _mega_body(idx_ref, x_ref, s_ref, w0_ref, b0_ref, wg0_ref, wc0_ref,
               w1_ref, b1_ref, wg1_ref, wc1_ref,
               wfc_ref, bfc_ref,
               o_ref, ss_ref, st0_ref, st1_ref, last_ref):
    t = pl.program_id(0)

    @pl.when(t == 0)
    def _():
        st0_ref[...] = jnp.zeros_like(st0_ref)
        st1_ref[...] = jnp.zeros_like(st1_ref)
        row = jax.lax.broadcasted_iota(jnp.int32, (N, N), 0)
        col = jax.lax.broadcasted_iota(jnp.int32, (N, N), 1)
        eye = (row == col).astype(F32)
        ss = s_ref[...]
        ss_ref[:N] = ss
        ss_ref[N:] = (2.0 * _dot(ss, ss) - eye).astype(BF16)

    sstack = ss_ref[...]

    h0_l0 = st0_ref[...]
    h0_l1 = st1_ref[...]
    h0b_l0 = h0_l0.astype(BF16)
    h0b_l1 = h0_l1.astype(BF16)
    x0 = x_ref[0]
    wide = jnp.concatenate([x0, h0b_l0, h0b_l1], axis=1)
    wd = _dot(sstack, wide)
    x1 = wd[:N, :B * D].astype(BF16)
    x2 = wd[N:, :B * D].astype(BF16)
    h1_l0 = wd[:N, B * D:2 * B * D].astype(BF16)
    h2_l0 = wd[N:, B * D:2 * B * D].astype(BF16)
    h1_l1 = wd[:N, 2 * B * D:].astype(BF16)
    h2_l1 = wd[N:, 2 * B * D:].astype(BF16)

    a0 = _proj_a(x0, x1, x2, w0_ref[...], b0_ref[0], D)
    new0 = _gru_tail(a0, sstack, wg0_ref[...], wc0_ref[...],
                     h0_l0, h0b_l0, h1_l0, h2_l0)
    x1in = jnp.concatenate([s.astype(BF16) for s in new0], axis=1)
    for bi in range(B):
        st0_ref[:, bi * H:(bi + 1) * H] = new0[bi]

    xd = _dot(sstack, x1in)
    y1 = xd[:N].astype(BF16)
    y2 = xd[N:].astype(BF16)
    a1 = _proj_a(x1in, y1, y2, w1_ref[...], b1_ref[0], H)
    new1 = _gru_tail(a1, sstack, wg1_ref[...], wc1_ref[...],
                     h0_l1, h0b_l1, h1_l1, h2_l1)
    for bi in range(B):
        sl = slice(bi * H, (bi + 1) * H)
        st1_ref[:, sl] = new1[bi]

        @pl.when(t == idx_ref[bi])
        def _(bi=bi, sl=sl):
            last_ref[:, sl] = new1[bi]

    @pl.when(t == T - 1)
    def _():
        wfc = wfc_ref[...]
        bfc = bfc_ref[0]
        for bi in range(B):
            sl = slice(bi * H, (bi + 1) * H)
            lg = _dot(jnp.maximum(last_ref[:, sl], 0.0), wfc) + bfc
            o_ref[bi:bi + 1, :] = jnp.max(lg, axis=0, keepdims=True)


def _mega(idx, x, s, w0_in, bias0, wg0_h, wc0_h, w1_in, bias1, wg1_h, wc1_h,
          wfc_pad, bfc_pad):
    return pl.pallas_call(
        _mega_body,
        grid=(T,),
        in_specs=[
            pl.BlockSpec(memory_space=pltpu.SMEM),
            pl.BlockSpec((1, N, B * D), lambda t: (t, 0, 0)),
            pl.BlockSpec((N, N), lambda t: (0, 0)),
            pl.BlockSpec((K * D, 3 * H), lambda t: (0, 0)),
            pl.BlockSpec((1, 3 * H), lambda t: (0, 0)),
            pl.BlockSpec((K * H, 2 * H), lambda t: (0, 0)),
            pl.BlockSpec((K * H, H), lambda t: (0, 0)),
            pl.BlockSpec((K * H, 3 * H), lambda t: (0, 0)),
            pl.BlockSpec((1, 3 * H), lambda t: (0, 0)),
            pl.BlockSpec((K * H, 2 * H), lambda t: (0, 0)),
            pl.BlockSpec((K * H, H), lambda t: (0, 0)),
            pl.BlockSpec((H, 128), lambda t: (0, 0)),
            pl.BlockSpec((1, 128), lambda t: (0, 0)),
        ],
        out_specs=pl.BlockSpec((B, 128), lambda t: (0, 0)),
        out_shape=jax.ShapeDtypeStruct((B, 128), F32),
        scratch_shapes=[
            pltpu.VMEM((2 * N, N), BF16),
            pltpu.VMEM((N, B * H), F32),
            pltpu.VMEM((N, B * H), F32),
            pltpu.VMEM((N, B * H), F32),
        ],
    )(idx, x, s, w0_in, bias0, wg0_h, wc0_h, w1_in, bias1, wg1_h, wc1_h,
      wfc_pad, bfc_pad)


def _split_weight(w, din, dout):
    wr = w.reshape(din + H, K, dout)
    w_in = wr[:din].transpose(1, 0, 2).reshape(K * din, dout)
    w_h = wr[din:].transpose(1, 0, 2).reshape(K * H, dout)
    return w_in, w_h


def kernel(input_seq, seq_lengths, supports, Wg0, bg0, Wc0, bc0,
           Wg1, bg1, Wc1, bc1, Wfc, bfc):
    s = supports[0].astype(BF16)

    wg0_in, wg0_h = _split_weight(Wg0, D, 2 * H)
    wc0_in, wc0_h = _split_weight(Wc0, D, H)
    wg1_in, wg1_h = _split_weight(Wg1, H, 2 * H)
    wc1_in, wc1_h = _split_weight(Wc1, H, H)
    w0_in = jnp.concatenate([wg0_in, wc0_in], axis=1).astype(BF16)
    w1_in = jnp.concatenate([wg1_in, wc1_in], axis=1).astype(BF16)
    wg0_h = wg0_h.astype(BF16)
    wc0_h = wc0_h.astype(BF16)
    wg1_h = wg1_h.astype(BF16)
    wc1_h = wc1_h.astype(BF16)
    bias0 = jnp.concatenate([bg0, bc0]).reshape(1, 3 * H)
    bias1 = jnp.concatenate([bg1, bc1]).reshape(1, 3 * H)

    idx = jnp.clip(seq_lengths - 1, 0, T - 1).astype(jnp.int32)

    wfc_pad = jnp.zeros((H, 128), BF16).at[:, :C].set(Wfc.astype(BF16))
    bfc_pad = jnp.zeros((1, 128), F32).at[0, :C].set(bfc)

    x0 = input_seq.transpose(1, 2, 0, 3).reshape(T, N, B * D).astype(BF16)
    logits_pad = _mega(idx, x0, s, w0_in, bias0, wg0_h, wc0_h,
                       w1_in, bias1, wg1_h, wc1_h, wfc_pad, bfc_pad)
    return logits_pad[:, :C]


# speedup vs baseline: 1.5132x; 1.0381x over previous
import jax
import jax.numpy as jnp
from jax.experimental import pallas as pl
from jax.experimental.pallas import tpu as pltpu

N = 512
D = 128
H = 128
T = 16
B = 4
K = 3
C = 4
F32 = jnp.float32
BF16 = jnp.bfloat16


def _dot(a, b):
    return jnp.dot(a.astype(BF16), b.astype(BF16),
                   preferred_element_type=F32)


def _proj_a(x0, x1, x2, w_in, bias, d):
    a_list = []
    for bi in range(B):
        sl = slice(bi * d, (bi + 1) * d)
        xc = jnp.concatenate([x0[:, sl], x1[:, sl], x2[:, sl]], axis=1)
        a_list.append(_dot(xc, w_in) + bias)
    return a_list


def _gru_tail(a_list, sstack, wg, wc, h0, h0b, h1, h2):
    us = []
    rs_parts = []
    for bi in range(B):
        sl = slice(bi * H, (bi + 1) * H)
        xc = jnp.concatenate([h0b[:, sl], h1[:, sl], h2[:, sl]], axis=1)
        g = jax.nn.sigmoid(a_list[bi][:, : 2 * H] + _dot(xc, wg))
        r, u = g[:, :H], g[:, H:]
        rs_parts.append((r * h0[:, sl]).astype(BF16))
        us.append(u)
    rs0 = jnp.concatenate(rs_parts, axis=1)
    rsd = _dot(sstack, rs0)
    rs1 = rsd[:N].astype(BF16)
    rs2 = rsd[N:].astype(BF16)
    new_states = []
    for bi in range(B):
        sl = slice(bi * H, (bi + 1) * H)
        xc = jnp.concatenate([rs0[:, sl], rs1[:, sl], rs2[:, sl]], axis=1)
        c = jnp.tanh(a_list[bi][:, 2 * H:] + _dot(xc, wc))
        u = us[bi]
        new_states.append(u * h0[:, sl] + (1.0 - u) * c)
    return new_states


def _mega_body(idx_ref, x_ref, s_ref, w0_ref, b0_ref, wg0_ref, wc0_ref,
               w1_ref, b1_ref, wg1_ref, wc1_ref,
               wfc_ref, bfc_ref,
               o_ref, ss_ref, st0_ref, st1_ref, last_ref):
    t = pl.program_id(0)

    @pl.when(t == 0)
    def _():
        st0_ref[...] = jnp.zeros_like(st0_ref)
        st1_ref[...] = jnp.zeros_like(st1_ref)
        row = jax.lax.broadcasted_iota(jnp.int32, (N, N), 0)
        col = jax.lax.broadcasted_iota(jnp.int32, (N, N), 1)
        eye = (row == col).astype(F32)
        ss = s_ref[...]
        ss_ref[:N] = ss
        ss_ref[N:] = (2.0 * _dot(ss, ss) - eye).astype(BF16)

    sstack = ss_ref[...]

    h0_l0 = st0_ref[...]
    h0_l1 = st1_ref[...]
    h0b_l0 = h0_l0.astype(BF16)
    h0b_l1 = h0_l1.astype(BF16)
    x0 = jnp.concatenate([x_ref[bi, 0].astype(BF16) for bi in range(B)],
                         axis=1)
    wide = jnp.concatenate([x0, h0b_l0, h0b_l1], axis=1)
    wd = _dot(sstack, wide)
    x1 = wd[:N, :B * D].astype(BF16)
    x2 = wd[N:, :B * D].astype(BF16)
    h1_l0 = wd[:N, B * D:2 * B * D].astype(BF16)
    h2_l0 = wd[N:, B * D:2 * B * D].astype(BF16)
    h1_l1 = wd[:N, 2 * B * D:].astype(BF16)
    h2_l1 = wd[N:, 2 * B * D:].astype(BF16)

    a0 = _proj_a(x0, x1, x2, w0_ref[...], b0_ref[0], D)
    new0 = _gru_tail(a0, sstack, wg0_ref[...], wc0_ref[...],
                     h0_l0, h0b_l0, h1_l0, h2_l0)
    x1in = jnp.concatenate([s.astype(BF16) for s in new0], axis=1)
    for bi in range(B):
        st0_ref[:, bi * H:(bi + 1) * H] = new0[bi]

    xd = _dot(sstack, x1in)
    y1 = xd[:N].astype(BF16)
    y2 = xd[N:].astype(BF16)
    a1 = _proj_a(x1in, y1, y2, w1_ref[...], b1_ref[0], H)
    new1 = _gru_tail(a1, sstack, wg1_ref[...], wc1_ref[...],
                     h0_l1, h0b_l1, h1_l1, h2_l1)
    for bi in range(B):
        sl = slice(bi * H, (bi + 1) * H)
        st1_ref[:, sl] = new1[bi]

        @pl.when(t == idx_ref[bi])
        def _(bi=bi, sl=sl):
            last_ref[:, sl] = new1[bi]

    @pl.when(t == T - 1)
    def _():
        wfc = wfc_ref[...]
        bfc = bfc_ref[0]
        for bi in range(B):
            sl = slice(bi * H, (bi + 1) * H)
            lg = _dot(jnp.maximum(last_ref[:, sl], 0.0), wfc) + bfc
            o_ref[bi:bi + 1, :] = jnp.max(lg, axis=0, keepdims=True)


def _mega(idx, x, s, w0_in, bias0, wg0_h, wc0_h, w1_in, bias1, wg1_h, wc1_h,
          wfc_pad, bfc_pad):
    return pl.pallas_call(
        _mega_body,
        grid=(T,),
        in_specs=[
            pl.BlockSpec(memory_space=pltpu.SMEM),
            pl.BlockSpec((B, 1, N, D), lambda t: (0, t, 0, 0)),
            pl.BlockSpec((N, N), lambda t: (0, 0)),
            pl.BlockSpec((K * D, 3 * H), lambda t: (0, 0)),
            pl.BlockSpec((1, 3 * H), lambda t: (0, 0)),
            pl.BlockSpec((K * H, 2 * H), lambda t: (0, 0)),
            pl.BlockSpec((K * H, H), lambda t: (0, 0)),
            pl.BlockSpec((K * H, 3 * H), lambda t: (0, 0)),
            pl.BlockSpec((1, 3 * H), lambda t: (0, 0)),
            pl.BlockSpec((K * H, 2 * H), lambda t: (0, 0)),
            pl.BlockSpec((K * H, H), lambda t: (0, 0)),
            pl.BlockSpec((H, 128), lambda t: (0, 0)),
            pl.BlockSpec((1, 128), lambda t: (0, 0)),
        ],
        out_specs=pl.BlockSpec((B, 128), lambda t: (0, 0)),
        out_shape=jax.ShapeDtypeStruct((B, 128), F32),
        scratch_shapes=[
            pltpu.VMEM((2 * N, N), BF16),
            pltpu.VMEM((N, B * H), F32),
            pltpu.VMEM((N, B * H), F32),
            pltpu.VMEM((N, B * H), F32),
        ],
    )(idx, x, s, w0_in, bias0, wg0_h, wc0_h, w1_in, bias1, wg1_h, wc1_h,
      wfc_pad, bfc_pad)


def _split_weight(w, din, dout):
    wr = w.reshape(din + H, K, dout)
    w_in = wr[:din].transpose(1, 0, 2).reshape(K * din, dout)
    w_h = wr[din:].transpose(1, 0, 2).reshape(K * H, dout)
    return w_in, w_h


def kernel(input_seq, seq_lengths, supports, Wg0, bg0, Wc0, bc0,
           Wg1, bg1, Wc1, bc1, Wfc, bfc):
    s = supports[0].astype(BF16)

    wg0_in, wg0_h = _split_weight(Wg0, D, 2 * H)
    wc0_in, wc0_h = _split_weight(Wc0, D, H)
    wg1_in, wg1_h = _split_weight(Wg1, H, 2 * H)
    wc1_in, wc1_h = _split_weight(Wc1, H, H)
    w0_in = jnp.concatenate([wg0_in, wc0_in], axis=1).astype(BF16)
    w1_in = jnp.concatenate([wg1_in, wc1_in], axis=1).astype(BF16)
    wg0_h = wg0_h.astype(BF16)
    wc0_h = wc0_h.astype(BF16)
    wg1_h = wg1_h.astype(BF16)
    wc1_h = wc1_h.astype(BF16)
    bias0 = jnp.concatenate([bg0, bc0]).reshape(1, 3 * H)
    bias1 = jnp.concatenate([bg1, bc1]).reshape(1, 3 * H)

    idx = jnp.clip(seq_lengths - 1, 0, T - 1).astype(jnp.int32)

    wfc_pad = jnp.zeros((H, 128), BF16).at[:, :C].set(Wfc.astype(BF16))
    bfc_pad = jnp.zeros((1, 128), F32).at[0, :C].set(bfc)

    logits_pad = _mega(idx, input_seq, s, w0_in, bias0, wg0_h, wc0_h,
                       w1_in, bias1, wg1_h, wc1_h, wfc_pad, bfc_pad)
    return logits_pad[:, :C]
